# Initial kernel scaffold; baseline (speedup 1.0000x reference)
import jax
import jax.numpy as jnp
from jax.experimental import pallas as pl


def kernel(pred, target):
    raise NotImplementedError("write your pallas kernel here")


# speedup vs baseline: 3.8302x; 3.8302x over previous
import math

import jax
import jax.numpy as jnp
from jax.experimental import pallas as pl
from jax.experimental.pallas import tpu as pltpu

_VOCAB = 32000
_PAD = 0
_SMOOTHING = 0.1
_CONF = 1.0 - _SMOOTHING
_EPS = _SMOOTHING / (_VOCAB - 2)
_K_CONST = _CONF * math.log(_CONF) + _SMOOTHING * math.log(_EPS)

_BR = 128
_BC = 3200


def _body(t_ref, x_ref, out_ref):
    i = pl.program_id(0)
    j = pl.program_id(1)

    @pl.when(jnp.logical_and(i == 0, j == 0))
    def _init():
        out_ref[0, 0] = 0.0

    x = x_ref[...]
    t = t_ref[0, 0, :]
    validf = (t != _PAD).astype(jnp.float32)

    s_rows = jnp.sum(jnp.sum(x, axis=1) * validf)

    col = jax.lax.broadcasted_iota(jnp.int32, (_BR, _BC), 1) + j * _BC
    hit = (col == t[:, None]) & (validf[:, None] > 0)
    s_gather = jnp.sum(jnp.where(hit, x, 0.0))

    s_first = jnp.sum(x[:, 0] * validf) * _EPS + jnp.sum(validf) * _K_CONST
    extra = jnp.where(j == 0, s_first, 0.0)

    out_ref[0, 0] += extra - _EPS * s_rows - (_CONF - _EPS) * s_gather


def kernel(pred, target):
    n = pred.shape[0]
    t3 = target.astype(jnp.int32).reshape(n // _BR, 1, _BR)
    out = pl.pallas_call(
        _body,
        grid=(n // _BR, _VOCAB // _BC),
        in_specs=[
            pl.BlockSpec((1, 1, _BR), lambda i, j: (i, 0, 0)),
            pl.BlockSpec((_BR, _BC), lambda i, j: (i, j)),
        ],
        out_specs=pl.BlockSpec(memory_space=pltpu.SMEM),
        out_shape=jax.ShapeDtypeStruct((1, 1), jnp.float32),
        compiler_params=pltpu.CompilerParams(
            dimension_semantics=("arbitrary", "arbitrary"),
        ),
    )(t3, pred)
    return out[0, 0]
